# Initial kernel scaffold; baseline (speedup 1.0000x reference)
import jax
import jax.numpy as jnp
from jax.experimental import pallas as pl


def kernel(node_features, W1, b1, W2, b2, edge_index, geo_ids):
    raise NotImplementedError("write your pallas kernel here")


# speedup vs baseline: 6.2809x; 6.2809x over previous
import functools

import jax
import jax.numpy as jnp
from jax import lax
from jax.experimental import pallas as pl
from jax.experimental.pallas import tpu as pltpu
from jax.experimental.pallas import tpu_sc as plsc

N = 50000
E = 500000
B = 16384
D_IN = 128
D_MID = 512
D_OUT = 2048

NP = 50176
NH = NP
EPAD = 524288
ROWS2D = EPAD // 128
NCORES = 2
NSUB = 16


def _sc_deg():
    rows_per_tile = ROWS2D // (NCORES * NSUB)
    ch_rows = 16
    nchunk = rows_per_tile // ch_rows
    mesh = plsc.VectorSubcoreMesh(core_axis_name="c", subcore_axis_name="s")

    @functools.partial(
        pl.kernel,
        out_type=jax.ShapeDtypeStruct((NCORES * NSUB, NH), jnp.float32),
        mesh=mesh,
        scratch_types=[
            pltpu.VMEM((ch_rows, 128), jnp.int32),
            pltpu.VMEM((NH,), jnp.float32),
        ],
        compiler_params=pltpu.CompilerParams(needs_layout_passes=False),
    )
    def kdeg(dst_hbm, out_hbm, idxb, hist):
        c = lax.axis_index("c")
        s = lax.axis_index("s")
        wid = c * NSUB + s

        def zbody(i, carry):
            hist[pl.ds(i * 16, 16)] = jnp.zeros((16,), jnp.float32)
            return carry
        lax.fori_loop(0, NH // 16, zbody, 0)

        ones = jnp.ones((16,), jnp.float32)

        def chunk(ch, carry):
            pltpu.sync_copy(
                dst_hbm.at[pl.ds(wid * rows_per_tile + ch * ch_rows, ch_rows)],
                idxb)
            for j in range(ch_rows):
                def inner(l, cy):
                    v = idxb[j, pl.ds(l * 16, 16)]
                    plsc.addupdate_scatter(hist, [v], ones)
                    return cy
                lax.fori_loop(0, 8, inner, 0)
            return carry
        lax.fori_loop(0, nchunk, chunk, 0)
        pltpu.sync_copy(hist, out_hbm.at[wid])

    return kdeg


def _sc_scatter(G):
    gpc = G // NCORES
    rows_per_tile = ROWS2D // NSUB
    ch_rows = 4
    nchunk = rows_per_tile // ch_rows
    init_per_tile = NP // NSUB
    mesh = plsc.VectorSubcoreMesh(core_axis_name="c", subcore_axis_name="s")

    @functools.partial(
        pl.kernel,
        out_type=jax.ShapeDtypeStruct((G * NP, 32), jnp.float32),
        mesh=mesh,
        scratch_types=[
            pltpu.VMEM((ch_rows, 128), jnp.int32),
            pltpu.VMEM((ch_rows, 128), jnp.int32),
            pltpu.VMEM((ch_rows * 128, 32), jnp.float32),
            pltpu.VMEM_SHARED((NH, 32), jnp.float32),
            pltpu.SemaphoreType.DMA,
        ],
        compiler_params=pltpu.CompilerParams(
            needs_layout_passes=False, use_tc_tiling_on_sc=False),
    )
    def kscat(table_hbm, src_hbm, dst_hbm, out_hbm, srcb, dstb, rows, acc, sem):
        c = lax.axis_index("c")
        s = lax.axis_index("s")
        for gi in range(gpc):
            goff = (c * gpc + gi) * NP
            pltpu.sync_copy(
                table_hbm.at[pl.ds(goff + s * init_per_tile, init_per_tile)],
                acc.at[pl.ds(s * init_per_tile, init_per_tile)])
            plsc.subcore_barrier()

            def chunk(ch, carry):
                roff = s * rows_per_tile + ch * ch_rows
                pltpu.sync_copy(src_hbm.at[pl.ds(roff, ch_rows)], srcb)
                pltpu.sync_copy(dst_hbm.at[pl.ds(roff, ch_rows)], dstb)
                for j in range(ch_rows):
                    def addoff(l, cy):
                        srcb[j, pl.ds(l * 16, 16)] = (
                            srcb[j, pl.ds(l * 16, 16)] + goff)
                        return cy
                    lax.fori_loop(0, 8, addoff, 0)
                cps = []
                for j in range(ch_rows):
                    cps.append(pltpu.async_copy(
                        table_hbm.at[srcb.at[j]],
                        rows.at[pl.ds(j * 128, 128)], sem))
                for cp in cps:
                    cp.wait()
                for j in range(ch_rows):
                    pltpu.sync_copy(rows.at[pl.ds(j * 128, 128)],
                                    acc.at[dstb.at[j]], add=True)
                return carry
            lax.fori_loop(0, nchunk, chunk, 0)
            plsc.subcore_barrier()
            pltpu.sync_copy(
                acc.at[pl.ds(s * init_per_tile, init_per_tile)],
                out_hbm.at[pl.ds(goff + s * init_per_tile, init_per_tile)])
            plsc.subcore_barrier()

    return kscat


def _sc_gather():
    per_tile = B // (NCORES * NSUB)
    idx_rows = per_tile // 128
    mesh = plsc.VectorSubcoreMesh(core_axis_name="c", subcore_axis_name="s")

    @functools.partial(
        pl.kernel,
        out_type=(jax.ShapeDtypeStruct((16, B, 32), jnp.float32),
                  jax.ShapeDtypeStruct((B, 32), jnp.float32)),
        mesh=mesh,
        scratch_types=[
            pltpu.VMEM((idx_rows, 128), jnp.int32),
            pltpu.VMEM((idx_rows, 128), jnp.int32),
            pltpu.VMEM((per_tile, 32), jnp.float32),
            pltpu.SemaphoreType.DMA,
        ],
        compiler_params=pltpu.CompilerParams(
            needs_layout_passes=False, use_tc_tiling_on_sc=False),
    )
    def kgat(t2_hbm, dinv_hbm, geo_hbm, t2geo_hbm, dgeo_hbm,
             geob, geoff, rows, sem):
        c = lax.axis_index("c")
        s = lax.axis_index("s")
        wid = c * NSUB + s
        base = wid * per_tile
        pltpu.sync_copy(geo_hbm.at[pl.ds(wid * idx_rows, idx_rows)], geob)
        for g in range(16):
            for j in range(idx_rows):
                def addoff(l, cy):
                    geoff[j, pl.ds(l * 16, 16)] = (
                        geob[j, pl.ds(l * 16, 16)] + g * NP)
                    return cy
                lax.fori_loop(0, 8, addoff, 0)
            cps = []
            for j in range(idx_rows):
                cps.append(pltpu.async_copy(
                    t2_hbm.at[geoff.at[j]],
                    rows.at[pl.ds(j * 128, 128)], sem))
            for cp in cps:
                cp.wait()
            pltpu.sync_copy(rows, t2geo_hbm.at[g].at[pl.ds(base, per_tile)])
        cps = []
        for j in range(idx_rows):
            cps.append(pltpu.async_copy(
                dinv_hbm.at[geob.at[j]],
                rows.at[pl.ds(j * 128, 128)], sem))
        for cp in cps:
            cp.wait()
        pltpu.sync_copy(rows, dgeo_hbm.at[pl.ds(base, per_tile)])

    return kgat


_BN = 1024
_BM = 512


def _prep_body(h_ref, x_ref, xsg_ref, dinv_ref):
    deg = jnp.sum(h_ref[...], axis=0) + 1.0
    dinv = lax.rsqrt(jnp.maximum(deg, 1e-12))[:, None]
    xs = x_ref[...] * dinv
    for g in range(4):
        xsg_ref[g] = xs[:, 32 * g:32 * (g + 1)]
    dinv_ref[...] = jnp.broadcast_to(dinv, (_BN, 32))


def _mm1_body(t1_ref, dinv_ref, w_ref, b_ref, y_ref):
    t = jnp.concatenate([t1_ref[g] for g in range(4)], axis=1)
    d = dinv_ref[...][:, 0:1]
    h = jnp.dot(t * d, w_ref[...], preferred_element_type=jnp.float32)
    y = jnp.maximum(h + b_ref[...], 0.0) * d
    for g in range(16):
        y_ref[g] = y[:, 32 * g:32 * (g + 1)]


def _mm2_body(t2_ref, dgeo_ref, w_ref, b_ref, o_ref):
    r = jnp.concatenate([t2_ref[g] for g in range(16)], axis=1)
    d = dgeo_ref[...][:, 0:1]
    z = jnp.dot(r * d, w_ref[...], preferred_element_type=jnp.float32)
    o_ref[...] = jnp.maximum(z + b_ref[...], 0.0).astype(jnp.bfloat16)


def _tc_prep(hists, x):
    return pl.pallas_call(
        _prep_body,
        grid=(NP // _BN,),
        in_specs=[
            pl.BlockSpec((NCORES * NSUB, _BN), lambda i: (0, i)),
            pl.BlockSpec((_BN, D_IN), lambda i: (i, 0)),
        ],
        out_specs=[
            pl.BlockSpec((4, _BN, 32), lambda i: (0, i, 0)),
            pl.BlockSpec((_BN, 32), lambda i: (i, 0)),
        ],
        out_shape=[
            jax.ShapeDtypeStruct((4, NP, 32), jnp.float32),
            jax.ShapeDtypeStruct((NP, 32), jnp.float32),
        ],
    )(hists, x)


def _tc_mm1(t1, dinv_rep, W1, b1r):
    return pl.pallas_call(
        _mm1_body,
        grid=(NP // _BN,),
        in_specs=[
            pl.BlockSpec((4, _BN, 32), lambda i: (0, i, 0)),
            pl.BlockSpec((_BN, 32), lambda i: (i, 0)),
            pl.BlockSpec((D_IN, D_MID), lambda i: (0, 0)),
            pl.BlockSpec((1, D_MID), lambda i: (0, 0)),
        ],
        out_specs=pl.BlockSpec((16, _BN, 32), lambda i: (0, i, 0)),
        out_shape=jax.ShapeDtypeStruct((16, NP, 32), jnp.float32),
    )(t1, dinv_rep, W1, b1r)


def _tc_mm2(t2geo, dgeo, W2, b2r):
    return pl.pallas_call(
        _mm2_body,
        grid=(B // _BM,),
        in_specs=[
            pl.BlockSpec((16, _BM, 32), lambda i: (0, i, 0)),
            pl.BlockSpec((_BM, 32), lambda i: (i, 0)),
            pl.BlockSpec((D_MID, D_OUT), lambda i: (0, 0)),
            pl.BlockSpec((1, D_OUT), lambda i: (0, 0)),
        ],
        out_specs=pl.BlockSpec((_BM, D_OUT), lambda i: (i, 0)),
        out_shape=jax.ShapeDtypeStruct((B, D_OUT), jnp.bfloat16),
    )(t2geo, dgeo, W2, b2r)


_deg_k = _sc_deg()
_scat4_k = _sc_scatter(4)
_scat16_k = _sc_scatter(16)
_gath_k = _sc_gather()


def kernel(node_features, W1, b1, W2, b2, edge_index, geo_ids):
    src = edge_index[0]
    dst = edge_index[1]
    pad = EPAD - E
    src2d = jnp.concatenate(
        [src, jnp.zeros((pad,), jnp.int32)]).reshape(ROWS2D, 128)
    dst2d = jnp.concatenate(
        [dst, jnp.full((pad,), N, jnp.int32)]).reshape(ROWS2D, 128)

    xp = jnp.concatenate(
        [node_features, jnp.zeros((NP - N, D_IN), jnp.float32)])
    hists = _deg_k(dst2d)
    xsg, dinv_rep = _tc_prep(hists, xp)
    t1 = _scat4_k(xsg.reshape(4 * NP, 32), src2d, dst2d)
    y1 = _tc_mm1(t1.reshape(4, NP, 32), dinv_rep, W1, b1.reshape(1, D_MID))
    t2 = _scat16_k(y1.reshape(16 * NP, 32), src2d, dst2d)
    t2geo, dgeo = _gath_k(t2, dinv_rep, geo_ids.reshape(128, 128))
    return _tc_mm2(t2geo, dgeo, W2, b2.reshape(1, D_OUT))


# speedup vs baseline: 7.5506x; 1.2022x over previous
import functools

import jax
import jax.numpy as jnp
from jax import lax
from jax.experimental import pallas as pl
from jax.experimental.pallas import tpu as pltpu
from jax.experimental.pallas import tpu_sc as plsc

N = 50000
E = 500000
B = 16384
D_IN = 128
D_MID = 512
D_OUT = 2048

NP = 50176
NH = NP
EPAD = 524288
ROWS2D = EPAD // 128
NCORES = 2
NSUB = 16


def _sc_deg():
    rows_per_tile = ROWS2D // (NCORES * NSUB)
    ch_rows = 16
    nchunk = rows_per_tile // ch_rows
    mesh = plsc.VectorSubcoreMesh(core_axis_name="c", subcore_axis_name="s")

    @functools.partial(
        pl.kernel,
        out_type=jax.ShapeDtypeStruct((NCORES * NSUB, NH), jnp.float32),
        mesh=mesh,
        scratch_types=[
            pltpu.VMEM((ch_rows, 128), jnp.int32),
            pltpu.VMEM((NH,), jnp.float32),
        ],
        compiler_params=pltpu.CompilerParams(needs_layout_passes=False),
    )
    def kdeg(dst_hbm, out_hbm, idxb, hist):
        c = lax.axis_index("c")
        s = lax.axis_index("s")
        wid = c * NSUB + s

        def zbody(i, carry):
            hist[pl.ds(i * 16, 16)] = jnp.zeros((16,), jnp.float32)
            return carry
        lax.fori_loop(0, NH // 16, zbody, 0)

        ones = jnp.ones((16,), jnp.float32)

        def chunk(ch, carry):
            pltpu.sync_copy(
                dst_hbm.at[pl.ds(wid * rows_per_tile + ch * ch_rows, ch_rows)],
                idxb)
            for j in range(ch_rows):
                def inner(l, cy):
                    v = idxb[j, pl.ds(l * 16, 16)]
                    plsc.addupdate_scatter(hist, [v], ones)
                    return cy
                lax.fori_loop(0, 8, inner, 0)
            return carry
        lax.fori_loop(0, nchunk, chunk, 0)
        pltpu.sync_copy(hist, out_hbm.at[wid])

    return kdeg


def _sc_scatter(G):
    gpc = G // NCORES
    rows_per_tile = ROWS2D // NSUB
    ch_rows = 2
    nchunk = rows_per_tile // ch_rows
    init_per_tile = NP // NSUB
    mesh = plsc.VectorSubcoreMesh(core_axis_name="c", subcore_axis_name="s")

    @functools.partial(
        pl.kernel,
        out_type=jax.ShapeDtypeStruct((G * NP, 32), jnp.float32),
        mesh=mesh,
        scratch_types=[
            pltpu.VMEM((2, ch_rows, 128), jnp.int32),
            pltpu.VMEM((2, ch_rows, 128), jnp.int32),
            pltpu.VMEM((2, ch_rows * 128, 32), jnp.float32),
            pltpu.VMEM_SHARED((NH, 32), jnp.float32),
            pltpu.SemaphoreType.DMA,
        ],
        compiler_params=pltpu.CompilerParams(
            needs_layout_passes=False, use_tc_tiling_on_sc=False),
    )
    def kscat(table_hbm, src_hbm, dst_hbm, out_hbm, srcb, dstb, rows, acc, sem):
        c = lax.axis_index("c")
        s = lax.axis_index("s")

        def load_fire(ch, p, goff):
            roff = s * rows_per_tile + ch * ch_rows
            pltpu.sync_copy(src_hbm.at[pl.ds(roff, ch_rows)], srcb.at[p])
            pltpu.sync_copy(dst_hbm.at[pl.ds(roff, ch_rows)], dstb.at[p])
            for j in range(ch_rows):
                def addoff(l, cy):
                    srcb[p, j, pl.ds(l * 16, 16)] = (
                        srcb[p, j, pl.ds(l * 16, 16)] + goff)
                    return cy
                lax.fori_loop(0, 8, addoff, 0)
            for j in range(ch_rows):
                pltpu.async_copy(table_hbm.at[srcb.at[p].at[j]],
                                 rows.at[p].at[pl.ds(j * 128, 128)], sem)

        def drain(p):
            for j in range(ch_rows):
                pltpu.make_async_copy(
                    table_hbm.at[srcb.at[p].at[j]],
                    rows.at[p].at[pl.ds(j * 128, 128)], sem).wait()

        def scatter(p):
            for j in range(ch_rows):
                pltpu.sync_copy(rows.at[p].at[pl.ds(j * 128, 128)],
                                acc.at[dstb.at[p].at[j]], add=True)

        for gi in range(gpc):
            goff = (c * gpc + gi) * NP
            pltpu.sync_copy(
                table_hbm.at[pl.ds(goff + s * init_per_tile, init_per_tile)],
                acc.at[pl.ds(s * init_per_tile, init_per_tile)])
            plsc.subcore_barrier()

            load_fire(0, 0, goff)

            def pair(i, carry):
                a = 2 * i
                load_fire(jnp.minimum(a + 1, nchunk - 1), 1, goff)
                drain(0)
                scatter(0)
                load_fire(jnp.minimum(a + 2, nchunk - 1), 0, goff)
                drain(1)
                scatter(1)
                return carry
            lax.fori_loop(0, nchunk // 2, pair, 0)
            drain(0)

            plsc.subcore_barrier()
            pltpu.sync_copy(
                acc.at[pl.ds(s * init_per_tile, init_per_tile)],
                out_hbm.at[pl.ds(goff + s * init_per_tile, init_per_tile)])
            plsc.subcore_barrier()

    return kscat


def _sc_gather():
    per_tile = B // (NCORES * NSUB)
    idx_rows = per_tile // 128
    mesh = plsc.VectorSubcoreMesh(core_axis_name="c", subcore_axis_name="s")

    @functools.partial(
        pl.kernel,
        out_type=(jax.ShapeDtypeStruct((16, B, 32), jnp.float32),
                  jax.ShapeDtypeStruct((B, 32), jnp.float32)),
        mesh=mesh,
        scratch_types=[
            pltpu.VMEM((idx_rows, 128), jnp.int32),
            pltpu.VMEM((idx_rows, 128), jnp.int32),
            pltpu.VMEM((per_tile, 32), jnp.float32),
            pltpu.SemaphoreType.DMA,
        ],
        compiler_params=pltpu.CompilerParams(
            needs_layout_passes=False, use_tc_tiling_on_sc=False),
    )
    def kgat(t2_hbm, dinv_hbm, geo_hbm, t2geo_hbm, dgeo_hbm,
             geob, geoff, rows, sem):
        c = lax.axis_index("c")
        s = lax.axis_index("s")
        wid = c * NSUB + s
        base = wid * per_tile
        pltpu.sync_copy(geo_hbm.at[pl.ds(wid * idx_rows, idx_rows)], geob)
        for g in range(16):
            for j in range(idx_rows):
                def addoff(l, cy):
                    geoff[j, pl.ds(l * 16, 16)] = (
                        geob[j, pl.ds(l * 16, 16)] + g * NP)
                    return cy
                lax.fori_loop(0, 8, addoff, 0)
            cps = []
            for j in range(idx_rows):
                cps.append(pltpu.async_copy(
                    t2_hbm.at[geoff.at[j]],
                    rows.at[pl.ds(j * 128, 128)], sem))
            for cp in cps:
                cp.wait()
            pltpu.sync_copy(rows, t2geo_hbm.at[g].at[pl.ds(base, per_tile)])
        cps = []
        for j in range(idx_rows):
            cps.append(pltpu.async_copy(
                dinv_hbm.at[geob.at[j]],
                rows.at[pl.ds(j * 128, 128)], sem))
        for cp in cps:
            cp.wait()
        pltpu.sync_copy(rows, dgeo_hbm.at[pl.ds(base, per_tile)])

    return kgat


_BN = 1024
_BM = 512


def _prep_body(h_ref, x_ref, xsg_ref, dinv_ref):
    deg = jnp.sum(h_ref[...], axis=0) + 1.0
    dinv = lax.rsqrt(jnp.maximum(deg, 1e-12))[:, None]
    xs = x_ref[...] * dinv
    for g in range(4):
        xsg_ref[g] = xs[:, 32 * g:32 * (g + 1)]
    dinv_ref[...] = jnp.broadcast_to(dinv, (_BN, 32))


def _mm1_body(t1_ref, dinv_ref, w_ref, b_ref, y_ref):
    t = jnp.concatenate([t1_ref[g] for g in range(4)], axis=1)
    d = dinv_ref[...][:, 0:1]
    h = jnp.dot(t * d, w_ref[...], preferred_element_type=jnp.float32)
    y = jnp.maximum(h + b_ref[...], 0.0) * d
    for g in range(16):
        y_ref[g] = y[:, 32 * g:32 * (g + 1)]


def _mm2_body(t2_ref, dgeo_ref, w_ref, b_ref, o_ref):
    r = jnp.concatenate([t2_ref[g] for g in range(16)], axis=1)
    d = dgeo_ref[...][:, 0:1]
    z = jnp.dot(r * d, w_ref[...], preferred_element_type=jnp.float32)
    o_ref[...] = jnp.maximum(z + b_ref[...], 0.0).astype(jnp.bfloat16)


def _tc_prep(hists, x):
    return pl.pallas_call(
        _prep_body,
        grid=(NP // _BN,),
        in_specs=[
            pl.BlockSpec((NCORES * NSUB, _BN), lambda i: (0, i)),
            pl.BlockSpec((_BN, D_IN), lambda i: (i, 0)),
        ],
        out_specs=[
            pl.BlockSpec((4, _BN, 32), lambda i: (0, i, 0)),
            pl.BlockSpec((_BN, 32), lambda i: (i, 0)),
        ],
        out_shape=[
            jax.ShapeDtypeStruct((4, NP, 32), jnp.float32),
            jax.ShapeDtypeStruct((NP, 32), jnp.float32),
        ],
    )(hists, x)


def _tc_mm1(t1, dinv_rep, W1, b1r):
    return pl.pallas_call(
        _mm1_body,
        grid=(NP // _BN,),
        in_specs=[
            pl.BlockSpec((4, _BN, 32), lambda i: (0, i, 0)),
            pl.BlockSpec((_BN, 32), lambda i: (i, 0)),
            pl.BlockSpec((D_IN, D_MID), lambda i: (0, 0)),
            pl.BlockSpec((1, D_MID), lambda i: (0, 0)),
        ],
        out_specs=pl.BlockSpec((16, _BN, 32), lambda i: (0, i, 0)),
        out_shape=jax.ShapeDtypeStruct((16, NP, 32), jnp.float32),
    )(t1, dinv_rep, W1, b1r)


def _tc_mm2(t2geo, dgeo, W2, b2r):
    return pl.pallas_call(
        _mm2_body,
        grid=(B // _BM,),
        in_specs=[
            pl.BlockSpec((16, _BM, 32), lambda i: (0, i, 0)),
            pl.BlockSpec((_BM, 32), lambda i: (i, 0)),
            pl.BlockSpec((D_MID, D_OUT), lambda i: (0, 0)),
            pl.BlockSpec((1, D_OUT), lambda i: (0, 0)),
        ],
        out_specs=pl.BlockSpec((_BM, D_OUT), lambda i: (i, 0)),
        out_shape=jax.ShapeDtypeStruct((B, D_OUT), jnp.bfloat16),
    )(t2geo, dgeo, W2, b2r)


_deg_k = _sc_deg()
_scat4_k = _sc_scatter(4)
_scat16_k = _sc_scatter(16)
_gath_k = _sc_gather()


def kernel(node_features, W1, b1, W2, b2, edge_index, geo_ids):
    src = edge_index[0]
    dst = edge_index[1]
    pad = EPAD - E
    src2d = jnp.concatenate(
        [src, jnp.zeros((pad,), jnp.int32)]).reshape(ROWS2D, 128)
    dst2d = jnp.concatenate(
        [dst, jnp.full((pad,), N, jnp.int32)]).reshape(ROWS2D, 128)

    xp = jnp.concatenate(
        [node_features, jnp.zeros((NP - N, D_IN), jnp.float32)])
    hists = _deg_k(dst2d)
    xsg, dinv_rep = _tc_prep(hists, xp)
    t1 = _scat4_k(xsg.reshape(4 * NP, 32), src2d, dst2d)
    y1 = _tc_mm1(t1.reshape(4, NP, 32), dinv_rep, W1, b1.reshape(1, D_MID))
    t2 = _scat16_k(y1.reshape(16 * NP, 32), src2d, dst2d)
    t2geo, dgeo = _gath_k(t2, dinv_rep, geo_ids.reshape(128, 128))
    return _tc_mm2(t2geo, dgeo, W2, b2.reshape(1, D_OUT))


# speedup vs baseline: 13.4962x; 1.7874x over previous
import functools

import jax
import jax.numpy as jnp
from jax import lax
from jax.experimental import pallas as pl
from jax.experimental.pallas import tpu as pltpu
from jax.experimental.pallas import tpu_sc as plsc

N = 50000
E = 500000
B = 16384
D_IN = 128
D_MID = 512
D_OUT = 2048

NP = 50176
NH = NP
EPAD = 524288
ROWS2D = EPAD // 128
NCORES = 2
NSUB = 16


def _sc_deg():
    rows_per_tile = ROWS2D // (NCORES * NSUB)
    ch_rows = 16
    nchunk = rows_per_tile // ch_rows
    mesh = plsc.VectorSubcoreMesh(core_axis_name="c", subcore_axis_name="s")

    @functools.partial(
        pl.kernel,
        out_type=jax.ShapeDtypeStruct((NCORES * NSUB, NH), jnp.float32),
        mesh=mesh,
        scratch_types=[
            pltpu.VMEM((ch_rows, 128), jnp.int32),
            pltpu.VMEM((NH,), jnp.float32),
        ],
        compiler_params=pltpu.CompilerParams(needs_layout_passes=False),
    )
    def kdeg(dst_hbm, out_hbm, idxb, hist):
        c = lax.axis_index("c")
        s = lax.axis_index("s")
        wid = c * NSUB + s

        def zbody(i, carry):
            hist[pl.ds(i * 16, 16)] = jnp.zeros((16,), jnp.float32)
            return carry
        lax.fori_loop(0, NH // 16, zbody, 0)

        ones = jnp.ones((16,), jnp.float32)

        def chunk(ch, carry):
            pltpu.sync_copy(
                dst_hbm.at[pl.ds(wid * rows_per_tile + ch * ch_rows, ch_rows)],
                idxb)
            for j in range(ch_rows):
                def inner(l, cy):
                    v = idxb[j, pl.ds(l * 16, 16)]
                    plsc.addupdate_scatter(hist, [v], ones)
                    return cy
                lax.fori_loop(0, 8, inner, 0)
            return carry
        lax.fori_loop(0, nchunk, chunk, 0)
        pltpu.sync_copy(hist, out_hbm.at[wid])

    return kdeg


def _sc_scatter(G):
    gpc = G // NCORES
    rows_per_tile = ROWS2D // NSUB
    ch_rows = 2
    nchunk = rows_per_tile // ch_rows
    init_per_tile = NP // NSUB
    mesh = plsc.VectorSubcoreMesh(core_axis_name="c", subcore_axis_name="s")

    @functools.partial(
        pl.kernel,
        out_type=jax.ShapeDtypeStruct((G * NP, 64), jnp.bfloat16),
        mesh=mesh,
        scratch_types=[
            pltpu.VMEM((2, ch_rows, 128), jnp.int32),
            pltpu.VMEM((2, ch_rows, 128), jnp.int32),
            pltpu.VMEM((2, ch_rows * 128, 64), jnp.bfloat16),
            pltpu.VMEM_SHARED((NH, 64), jnp.bfloat16),
            pltpu.SemaphoreType.DMA,
        ],
        compiler_params=pltpu.CompilerParams(
            needs_layout_passes=False, use_tc_tiling_on_sc=False),
    )
    def kscat(table_hbm, src_hbm, dst_hbm, out_hbm, srcb, dstb, rows, acc, sem):
        c = lax.axis_index("c")
        s = lax.axis_index("s")

        def load_fire(ch, p, goff):
            roff = s * rows_per_tile + ch * ch_rows
            pltpu.sync_copy(src_hbm.at[pl.ds(roff, ch_rows)], srcb.at[p])
            pltpu.sync_copy(dst_hbm.at[pl.ds(roff, ch_rows)], dstb.at[p])
            for j in range(ch_rows):
                def addoff(l, cy):
                    srcb[p, j, pl.ds(l * 16, 16)] = (
                        srcb[p, j, pl.ds(l * 16, 16)] + goff)
                    return cy
                lax.fori_loop(0, 8, addoff, 0)
            for j in range(ch_rows):
                pltpu.async_copy(table_hbm.at[srcb.at[p].at[j]],
                                 rows.at[p].at[pl.ds(j * 128, 128)], sem)

        def drain(p):
            for j in range(ch_rows):
                pltpu.make_async_copy(
                    table_hbm.at[srcb.at[p].at[j]],
                    rows.at[p].at[pl.ds(j * 128, 128)], sem).wait()

        def scatter(p):
            for j in range(ch_rows):
                pltpu.sync_copy(rows.at[p].at[pl.ds(j * 128, 128)],
                                acc.at[dstb.at[p].at[j]], add=True)

        for gi in range(gpc):
            goff = (c * gpc + gi) * NP
            pltpu.sync_copy(
                table_hbm.at[pl.ds(goff + s * init_per_tile, init_per_tile)],
                acc.at[pl.ds(s * init_per_tile, init_per_tile)])
            plsc.subcore_barrier()

            load_fire(0, 0, goff)

            def pair(i, carry):
                a = 2 * i
                load_fire(jnp.minimum(a + 1, nchunk - 1), 1, goff)
                drain(0)
                scatter(0)
                load_fire(jnp.minimum(a + 2, nchunk - 1), 0, goff)
                drain(1)
                scatter(1)
                return carry
            lax.fori_loop(0, nchunk // 2, pair, 0)
            drain(0)

            plsc.subcore_barrier()
            pltpu.sync_copy(
                acc.at[pl.ds(s * init_per_tile, init_per_tile)],
                out_hbm.at[pl.ds(goff + s * init_per_tile, init_per_tile)])
            plsc.subcore_barrier()

    return kscat


def _sc_gather():
    per_tile = B // (NCORES * NSUB)
    idx_rows = per_tile // 128
    mesh = plsc.VectorSubcoreMesh(core_axis_name="c", subcore_axis_name="s")

    @functools.partial(
        pl.kernel,
        out_type=(jax.ShapeDtypeStruct((8, B, 64), jnp.bfloat16),
                  jax.ShapeDtypeStruct((B, 32), jnp.float32)),
        mesh=mesh,
        scratch_types=[
            pltpu.VMEM((idx_rows, 128), jnp.int32),
            pltpu.VMEM((idx_rows, 128), jnp.int32),
            pltpu.VMEM((per_tile, 64), jnp.bfloat16),
            pltpu.VMEM((per_tile, 32), jnp.float32),
            pltpu.SemaphoreType.DMA,
        ],
        compiler_params=pltpu.CompilerParams(
            needs_layout_passes=False, use_tc_tiling_on_sc=False),
    )
    def kgat(t2_hbm, dinv_hbm, geo_hbm, t2geo_hbm, dgeo_hbm,
             geob, geoff, rows, drows, sem):
        c = lax.axis_index("c")
        s = lax.axis_index("s")
        wid = c * NSUB + s
        base = wid * per_tile
        pltpu.sync_copy(geo_hbm.at[pl.ds(wid * idx_rows, idx_rows)], geob)
        for g in range(8):
            for j in range(idx_rows):
                def addoff(l, cy):
                    geoff[j, pl.ds(l * 16, 16)] = (
                        geob[j, pl.ds(l * 16, 16)] + g * NP)
                    return cy
                lax.fori_loop(0, 8, addoff, 0)
            cps = []
            for j in range(idx_rows):
                cps.append(pltpu.async_copy(
                    t2_hbm.at[geoff.at[j]],
                    rows.at[pl.ds(j * 128, 128)], sem))
            for cp in cps:
                cp.wait()
            pltpu.sync_copy(rows, t2geo_hbm.at[g].at[pl.ds(base, per_tile)])
        cps = []
        for j in range(idx_rows):
            cps.append(pltpu.async_copy(
                dinv_hbm.at[geob.at[j]],
                drows.at[pl.ds(j * 128, 128)], sem))
        for cp in cps:
            cp.wait()
        pltpu.sync_copy(drows, dgeo_hbm.at[pl.ds(base, per_tile)])

    return kgat


_BN = 1024
_BM = 512


def _prep_body(h_ref, x_ref, xsg_ref, dinv_ref):
    deg = jnp.sum(h_ref[...], axis=0) + 1.0
    dinv = lax.rsqrt(jnp.maximum(deg, 1e-12))[:, None]
    xs = (x_ref[...] * dinv).astype(jnp.bfloat16)
    for g in range(2):
        xsg_ref[g] = xs[:, 64 * g:64 * (g + 1)]
    dinv_ref[...] = jnp.broadcast_to(dinv, (_BN, 32))


def _mm1_body(t1_ref, dinv_ref, w_ref, b_ref, y_ref):
    t = jnp.concatenate(
        [t1_ref[g] for g in range(2)], axis=1).astype(jnp.float32)
    d = dinv_ref[...][:, 0:1]
    h = jnp.dot(t * d, w_ref[...], preferred_element_type=jnp.float32)
    y = (jnp.maximum(h + b_ref[...], 0.0) * d).astype(jnp.bfloat16)
    for g in range(8):
        y_ref[g] = y[:, 64 * g:64 * (g + 1)]


def _mm2_body(t2_ref, dgeo_ref, w_ref, b_ref, o_ref):
    r = jnp.concatenate(
        [t2_ref[g] for g in range(8)], axis=1).astype(jnp.float32)
    d = dgeo_ref[...][:, 0:1]
    z = jnp.dot(r * d, w_ref[...], preferred_element_type=jnp.float32)
    o_ref[...] = jnp.maximum(z + b_ref[...], 0.0).astype(jnp.bfloat16)


def _tc_prep(hists, x):
    return pl.pallas_call(
        _prep_body,
        grid=(NP // _BN,),
        in_specs=[
            pl.BlockSpec((NCORES * NSUB, _BN), lambda i: (0, i)),
            pl.BlockSpec((_BN, D_IN), lambda i: (i, 0)),
        ],
        out_specs=[
            pl.BlockSpec((2, _BN, 64), lambda i: (0, i, 0)),
            pl.BlockSpec((_BN, 32), lambda i: (i, 0)),
        ],
        out_shape=[
            jax.ShapeDtypeStruct((2, NP, 64), jnp.bfloat16),
            jax.ShapeDtypeStruct((NP, 32), jnp.float32),
        ],
    )(hists, x)


def _tc_mm1(t1, dinv_rep, W1, b1r):
    return pl.pallas_call(
        _mm1_body,
        grid=(NP // _BN,),
        in_specs=[
            pl.BlockSpec((2, _BN, 64), lambda i: (0, i, 0)),
            pl.BlockSpec((_BN, 32), lambda i: (i, 0)),
            pl.BlockSpec((D_IN, D_MID), lambda i: (0, 0)),
            pl.BlockSpec((1, D_MID), lambda i: (0, 0)),
        ],
        out_specs=pl.BlockSpec((8, _BN, 64), lambda i: (0, i, 0)),
        out_shape=jax.ShapeDtypeStruct((8, NP, 64), jnp.bfloat16),
    )(t1, dinv_rep, W1, b1r)


def _tc_mm2(t2geo, dgeo, W2, b2r):
    return pl.pallas_call(
        _mm2_body,
        grid=(B // _BM,),
        in_specs=[
            pl.BlockSpec((8, _BM, 64), lambda i: (0, i, 0)),
            pl.BlockSpec((_BM, 32), lambda i: (i, 0)),
            pl.BlockSpec((D_MID, D_OUT), lambda i: (0, 0)),
            pl.BlockSpec((1, D_OUT), lambda i: (0, 0)),
        ],
        out_specs=pl.BlockSpec((_BM, D_OUT), lambda i: (i, 0)),
        out_shape=jax.ShapeDtypeStruct((B, D_OUT), jnp.bfloat16),
    )(t2geo, dgeo, W2, b2r)


_deg_k = _sc_deg()
_scat2_k = _sc_scatter(2)
_scat8_k = _sc_scatter(8)
_gath_k = _sc_gather()


def kernel(node_features, W1, b1, W2, b2, edge_index, geo_ids):
    src = edge_index[0]
    dst = edge_index[1]
    pad = EPAD - E
    src2d = jnp.concatenate(
        [src, jnp.zeros((pad,), jnp.int32)]).reshape(ROWS2D, 128)
    dst2d = jnp.concatenate(
        [dst, jnp.full((pad,), N, jnp.int32)]).reshape(ROWS2D, 128)

    xp = jnp.concatenate(
        [node_features, jnp.zeros((NP - N, D_IN), jnp.float32)])
    hists = _deg_k(dst2d)
    xsg, dinv_rep = _tc_prep(hists, xp)
    t1 = _scat2_k(xsg.reshape(2 * NP, 64), src2d, dst2d)
    y1 = _tc_mm1(t1.reshape(2, NP, 64), dinv_rep, W1, b1.reshape(1, D_MID))
    t2 = _scat8_k(y1.reshape(8 * NP, 64), src2d, dst2d)
    t2geo, dgeo = _gath_k(t2, dinv_rep, geo_ids.reshape(128, 128))
    return _tc_mm2(t2geo, dgeo, W2, b2.reshape(1, D_OUT))
